# baseline (speedup 1.0000x reference)
import functools

import jax
import jax.numpy as jnp
from jax import lax
from jax.experimental import pallas as pl
from jax.experimental.pallas import tpu as pltpu
import jax.experimental.pallas.tpu_sc as plsc

NUM_CLASSES_ = 100001
ALPHA_ = 0.25
EPS_ = 1e-7

B_ = 1024
L_ = 50

NUM_WORKERS = 32
IDX_TOTAL = B_ * L_
IDX_PER_W = IDX_TOTAL // NUM_WORKERS
IDX_CHUNK = 128
IDX_PAD = ((IDX_PER_W + IDX_CHUNK - 1) // IDX_CHUNK) * IDX_CHUNK
N_CHUNKS = IDX_PAD // IDX_CHUNK
VREGS_PER_W = IDX_PER_W // 16

ROW_BLK = 256
COL_BLK = 4096


def _sc_gather_kernel(labels_hbm, pred_hbm, out_hbm, lab_v, idx_v, rows_v, sem):
    wid = lax.axis_index("s") * 2 + lax.axis_index("c")
    base = wid * IDX_PER_W
    pltpu.sync_copy(labels_hbm.at[pl.ds(base, IDX_PER_W)], lab_v)

    def body(n, carry):
        off = n * 16
        lab = lab_v[pl.ds(off, 16)]
        pos = base + off + lax.broadcasted_iota(jnp.int32, (16,), 0)
        row = lax.div(pos, L_)
        idx_v[pl.ds(off, 16)] = lab + row * NUM_CLASSES_
        return carry

    lax.fori_loop(0, VREGS_PER_W, body, 0)
    zeros16 = jnp.zeros((16,), jnp.int32)
    for t in range((IDX_PAD - IDX_PER_W) // 16):
        idx_v[pl.ds(IDX_PER_W + t * 16, 16)] = zeros16

    copies = []
    for c in range(N_CHUNKS):
        copies.append(
            pltpu.async_copy(
                pred_hbm.at[idx_v.at[pl.ds(c * IDX_CHUNK, IDX_CHUNK)]],
                rows_v.at[pl.ds(c * IDX_CHUNK, IDX_CHUNK)],
                sem,
            )
        )
    for cp in copies:
        cp.wait()
    pltpu.sync_copy(rows_v.at[pl.ds(0, IDX_PER_W)], out_hbm.at[pl.ds(base, IDX_PER_W)])


@functools.cache
def _sc_gather():
    return pl.kernel(
        _sc_gather_kernel,
        out_type=jax.ShapeDtypeStruct((IDX_TOTAL,), jnp.float32),
        mesh=plsc.VectorSubcoreMesh(core_axis_name="c", subcore_axis_name="s"),
        scratch_types=[
            pltpu.VMEM((IDX_PER_W,), jnp.int32),
            pltpu.VMEM((IDX_PAD,), jnp.int32),
            pltpu.VMEM((IDX_PAD,), jnp.float32),
            pltpu.SemaphoreType.DMA,
        ],
    )


def _fneg(p):
    return (1.0 - ALPHA_) * p * p * (-jnp.log(1.0 - p))


def _fpos(p):
    q = 1.0 - p
    return ALPHA_ * q * q * (-jnp.log(p))


def _corr_kernel(labels_ref, gath_ref, corr_ref):
    labels = labels_ref[...]
    g = jnp.clip(gath_ref[...], EPS_, 1.0 - EPS_)
    eq = labels[:, :, None] == labels[:, None, :]
    kk = lax.broadcasted_iota(jnp.int32, (L_, L_), 1)
    jj = lax.broadcasted_iota(jnp.int32, (L_, L_), 0)
    earlier = (kk < jj)[None, :, :]
    dup = jnp.any(eq & earlier, axis=2)
    mask = jnp.where((labels > 0) & ~dup, 1.0, 0.0)
    delta = mask * (_fpos(g) - _fneg(g))
    corr_ref[...] = jnp.sum(delta, axis=1, keepdims=True)


def _corr(labels, gathered):
    return pl.pallas_call(
        _corr_kernel,
        grid=(B_ // ROW_BLK,),
        in_specs=[
            pl.BlockSpec((ROW_BLK, L_), lambda i: (i, 0)),
            pl.BlockSpec((ROW_BLK, L_), lambda i: (i, 0)),
        ],
        out_specs=pl.BlockSpec((ROW_BLK, 1), lambda i: (i, 0)),
        out_shape=jax.ShapeDtypeStruct((B_, 1), jnp.float32),
    )(labels, gathered)


N_COL = (NUM_CLASSES_ + COL_BLK - 1) // COL_BLK


def _tc_kernel(pred_ref, corr_ref, out_ref):
    j = pl.program_id(1)
    p = jnp.clip(pred_ref[...], EPS_, 1.0 - EPS_)
    fneg = _fneg(p)

    @pl.when(j == N_COL - 1)
    def _masked():
        col = j * COL_BLK + lax.broadcasted_iota(jnp.int32, (ROW_BLK, COL_BLK), 1)
        partial = jnp.sum(jnp.where(col < NUM_CLASSES_, fneg, 0.0), axis=1, keepdims=True)
        out_ref[...] += partial

    @pl.when(j == 0)
    def _init():
        out_ref[...] = corr_ref[...] + jnp.sum(fneg, axis=1, keepdims=True)

    @pl.when((j != 0) & (j != N_COL - 1))
    def _acc():
        out_ref[...] += jnp.sum(fneg, axis=1, keepdims=True)


def _tc_focal(predictions, corr):
    grid = (B_ // ROW_BLK, N_COL)
    return pl.pallas_call(
        _tc_kernel,
        grid=grid,
        in_specs=[
            pl.BlockSpec((ROW_BLK, COL_BLK), lambda i, j: (i, j)),
            pl.BlockSpec((ROW_BLK, 1), lambda i, j: (i, 0)),
        ],
        out_specs=pl.BlockSpec((ROW_BLK, 1), lambda i, j: (i, 0)),
        out_shape=jax.ShapeDtypeStruct((B_, 1), jnp.float32),
        compiler_params=pltpu.CompilerParams(
            dimension_semantics=("arbitrary", "arbitrary"),
        ),
    )(predictions, corr)


def kernel(predictions, labels):
    labels = labels.astype(jnp.int32)
    pred_flat = predictions.reshape(B_ * NUM_CLASSES_)
    labels_flat = labels.reshape(IDX_TOTAL)
    gathered = _sc_gather()(labels_flat, pred_flat).reshape(B_, L_)
    corr = _corr(labels, gathered)
    loss = _tc_focal(predictions, corr)
    return loss.reshape(B_)


# speedup vs baseline: 5.1462x; 5.1462x over previous
import jax
import jax.numpy as jnp
from jax import lax
from jax.experimental import pallas as pl
from jax.experimental.pallas import tpu as pltpu

NUM_CLASSES_ = 100001
ALPHA_ = 0.25
EPS_ = 1e-7

B_ = 1024
L_ = 50

ROW_BLK = 256
COL_BLK = 4096
N_COL = (NUM_CLASSES_ + COL_BLK - 1) // COL_BLK
N_GRP = COL_BLK // 128


def _fneg(p):
    return (1.0 - ALPHA_) * p * p * (-jnp.log(1.0 - p))


def _fpos(p):
    q = 1.0 - p
    return ALPHA_ * q * q * (-jnp.log(p))


def _mask_kernel(labels_ref, mlab_ref):
    labels = labels_ref[...]
    eq = labels[:, :, None] == labels[:, None, :]
    kk = lax.broadcasted_iota(jnp.int32, (L_, L_), 1)
    jj = lax.broadcasted_iota(jnp.int32, (L_, L_), 0)
    earlier = (kk < jj)[None, :, :]
    dup = jnp.any(eq & earlier, axis=2)
    mlab_ref[...] = jnp.where((labels > 0) & ~dup, labels, -1)


def _mask_labels(labels):
    return pl.pallas_call(
        _mask_kernel,
        grid=(B_ // ROW_BLK,),
        in_specs=[pl.BlockSpec((ROW_BLK, L_), lambda i: (i, 0))],
        out_specs=pl.BlockSpec((ROW_BLK, L_), lambda i: (i, 0)),
        out_shape=jax.ShapeDtypeStruct((B_, L_), jnp.int32),
    )(labels)


def _tc_kernel(pred_ref, mlab_ref, out_ref):
    j = pl.program_id(1)
    p = jnp.clip(pred_ref[...], EPS_, 1.0 - EPS_)
    fneg = _fneg(p)

    @pl.when(j == N_COL - 1)
    def _masked():
        col = j * COL_BLK + lax.broadcasted_iota(jnp.int32, (ROW_BLK, COL_BLK), 1)
        dense = jnp.sum(jnp.where(col < NUM_CLASSES_, fneg, 0.0), axis=1, keepdims=True)
        out_ref[...] += dense

    @pl.when(j != N_COL - 1)
    def _dense():
        partial = jnp.sum(fneg, axis=1, keepdims=True)

        @pl.when(j == 0)
        def _init():
            out_ref[...] = partial

        @pl.when(j != 0)
        def _acc():
            out_ref[...] += partial

    local = mlab_ref[...] - j * COL_BLK
    g_blk = jnp.zeros((ROW_BLK, L_), jnp.float32)
    vmask = jnp.zeros((ROW_BLK, L_), jnp.bool_)
    for s in range(N_GRP):
        idx_s = local - s * 128
        valid_s = (idx_s >= 0) & (idx_s < 128)
        g_s = jnp.take_along_axis(p[:, s * 128:(s + 1) * 128], idx_s & 127, axis=1)
        g_blk = jnp.where(valid_s, g_s, g_blk)
        vmask = vmask | valid_s
    g_c = jnp.clip(g_blk, EPS_, 1.0 - EPS_)
    delta = jnp.where(vmask, _fpos(g_c) - _fneg(g_c), 0.0)
    out_ref[...] += jnp.sum(delta, axis=1, keepdims=True)


def _tc_focal(predictions, mlab):
    grid = (B_ // ROW_BLK, N_COL)
    return pl.pallas_call(
        _tc_kernel,
        grid=grid,
        in_specs=[
            pl.BlockSpec((ROW_BLK, COL_BLK), lambda i, j: (i, j)),
            pl.BlockSpec((ROW_BLK, L_), lambda i, j: (i, 0)),
        ],
        out_specs=pl.BlockSpec((ROW_BLK, 1), lambda i, j: (i, 0)),
        out_shape=jax.ShapeDtypeStruct((B_, 1), jnp.float32),
        compiler_params=pltpu.CompilerParams(
            dimension_semantics=("arbitrary", "arbitrary"),
        ),
    )(predictions, mlab)


def kernel(predictions, labels):
    labels = labels.astype(jnp.int32)
    mlab = _mask_labels(labels)
    loss = _tc_focal(predictions, mlab)
    return loss.reshape(B_)


# speedup vs baseline: 5.5542x; 1.0793x over previous
import jax
import jax.numpy as jnp
from jax import lax
from jax.experimental import pallas as pl
from jax.experimental.pallas import tpu as pltpu

NUM_CLASSES_ = 100001
ALPHA_ = 0.25
EPS_ = 1e-7

B_ = 1024
L_ = 50

ROW_BLK = 256
COL_BLK = 4096
N_COL = (NUM_CLASSES_ + COL_BLK - 1) // COL_BLK
N_GRP = COL_BLK // 128


def _fneg(p):
    return (1.0 - ALPHA_) * p * p * (-jnp.log(1.0 - p))


def _fpos(p):
    q = 1.0 - p
    return ALPHA_ * q * q * (-jnp.log(p))


def _mask_kernel(labels_ref, mlab_ref):
    labels = labels_ref[...]
    eq = labels[:, :, None] == labels[:, None, :]
    kk = lax.broadcasted_iota(jnp.int32, (L_, L_), 1)
    jj = lax.broadcasted_iota(jnp.int32, (L_, L_), 0)
    earlier = (kk < jj)[None, :, :]
    dup = jnp.any(eq & earlier, axis=2)
    mlab_ref[...] = jnp.where((labels > 0) & ~dup, labels, -1)


def _mask_labels(labels):
    return pl.pallas_call(
        _mask_kernel,
        grid=(B_ // ROW_BLK,),
        in_specs=[pl.BlockSpec((ROW_BLK, L_), lambda i: (i, 0))],
        out_specs=pl.BlockSpec((ROW_BLK, L_), lambda i: (i, 0)),
        out_shape=jax.ShapeDtypeStruct((B_, L_), jnp.int32),
    )(labels)


def _tc_kernel(pred_ref, mlab_ref, out_ref):
    j = pl.program_id(1)
    p = jnp.clip(pred_ref[...], EPS_, 1.0 - EPS_)
    fneg = _fneg(p)

    @pl.when(j == N_COL - 1)
    def _masked():
        col = j * COL_BLK + lax.broadcasted_iota(jnp.int32, (ROW_BLK, COL_BLK), 1)
        dense = jnp.sum(jnp.where(col < NUM_CLASSES_, fneg, 0.0), axis=1, keepdims=True)
        out_ref[...] += dense

    @pl.when(j != N_COL - 1)
    def _dense():
        partial = jnp.sum(fneg, axis=1, keepdims=True)

        @pl.when(j == 0)
        def _init():
            out_ref[...] = partial

        @pl.when(j != 0)
        def _acc():
            out_ref[...] += partial

    local = mlab_ref[...] - j * COL_BLK
    hi = lax.shift_right_arithmetic(local, 7)
    lo = local & 127
    g_blk = jnp.zeros((ROW_BLK, L_), jnp.float32)
    for s in range(N_GRP):
        g_s = jnp.take_along_axis(p[:, s * 128:(s + 1) * 128], lo, axis=1)
        g_blk = jnp.where(hi == s, g_s, g_blk)
    vmask = (hi >= 0) & (hi < N_GRP)
    g_c = jnp.clip(g_blk, EPS_, 1.0 - EPS_)
    delta = jnp.where(vmask, _fpos(g_c) - _fneg(g_c), 0.0)
    out_ref[...] += jnp.sum(delta, axis=1, keepdims=True)


def _tc_focal(predictions, mlab):
    grid = (B_ // ROW_BLK, N_COL)
    return pl.pallas_call(
        _tc_kernel,
        grid=grid,
        in_specs=[
            pl.BlockSpec((ROW_BLK, COL_BLK), lambda i, j: (i, j)),
            pl.BlockSpec((ROW_BLK, L_), lambda i, j: (i, 0)),
        ],
        out_specs=pl.BlockSpec((ROW_BLK, 1), lambda i, j: (i, 0)),
        out_shape=jax.ShapeDtypeStruct((B_, 1), jnp.float32),
        compiler_params=pltpu.CompilerParams(
            dimension_semantics=("parallel", "arbitrary"),
        ),
    )(predictions, mlab)


def kernel(predictions, labels):
    labels = labels.astype(jnp.int32)
    mlab = _mask_labels(labels)
    loss = _tc_focal(predictions, mlab)
    return loss.reshape(B_)
